# Initial kernel scaffold; baseline (speedup 1.0000x reference)
import jax
import jax.numpy as jnp
from jax.experimental import pallas as pl


def kernel(x, gate_W, gate_b, expert_W, expert_b):
    raise NotImplementedError("write your pallas kernel here")


# speedup vs baseline: 1.1012x; 1.1012x over previous
import functools

import jax
import jax.numpy as jnp
from jax.experimental import pallas as pl
from jax.experimental.pallas import tpu as pltpu

TOP_K = 2


def _router_body(x_ref, gw_ref, gb_ref, logits_ref, sel_ref, w_ref, wmat_ref):
    E = gw_ref.shape[1]
    x = x_ref[...]
    lg = jnp.dot(x, gw_ref[...], preferred_element_type=jnp.float32)
    lg = lg + gb_ref[...]
    logits_ref[...] = lg
    m = jnp.max(lg, axis=-1, keepdims=True)
    ex = jnp.exp(lg - m)
    p = ex / jnp.sum(ex, axis=-1, keepdims=True)
    iota = jax.lax.broadcasted_iota(jnp.int32, p.shape, 1)
    m1 = jnp.max(p, axis=-1, keepdims=True)
    i1 = jnp.min(jnp.where(p == m1, iota, E), axis=-1, keepdims=True)
    pm = jnp.where(iota == i1, -jnp.inf, p)
    m2 = jnp.max(pm, axis=-1, keepdims=True)
    i2 = jnp.min(jnp.where(pm == m2, iota, E), axis=-1, keepdims=True)
    s = m1 + m2
    w1 = m1 / s
    w2 = m2 / s
    sel_ref[...] = jnp.concatenate([i1, i2], axis=1)
    w_ref[...] = jnp.concatenate([w1, w2], axis=1)
    wmat_ref[...] = jnp.where(iota == i1, w1, 0.0) + jnp.where(iota == i2, w2, 0.0)


def _router(x2, gate_W, gate_b, bm):
    T, d = x2.shape
    E = gate_W.shape[1]
    grid = (T // bm,)
    return pl.pallas_call(
        _router_body,
        grid=grid,
        in_specs=[
            pl.BlockSpec((bm, d), lambda m: (m, 0)),
            pl.BlockSpec((d, E), lambda m: (0, 0)),
            pl.BlockSpec((1, E), lambda m: (0, 0)),
        ],
        out_specs=[
            pl.BlockSpec((bm, E), lambda m: (m, 0)),
            pl.BlockSpec((bm, TOP_K), lambda m: (m, 0)),
            pl.BlockSpec((bm, TOP_K), lambda m: (m, 0)),
            pl.BlockSpec((bm, E), lambda m: (m, 0)),
        ],
        out_shape=[
            jax.ShapeDtypeStruct((T, E), jnp.float32),
            jax.ShapeDtypeStruct((T, TOP_K), jnp.int32),
            jax.ShapeDtypeStruct((T, TOP_K), jnp.float32),
            jax.ShapeDtypeStruct((T, E), jnp.float32),
        ],
    )(x2, gate_W, gate_b.reshape(1, E))


def _dense_body(x_ref, w_ref, b_ref, wmat_ref, out_ref, acc_ref):
    e = pl.program_id(2)
    ne = pl.num_programs(2)

    @pl.when(e == 0)
    def _():
        acc_ref[...] = jnp.zeros_like(acc_ref)

    y = jnp.dot(x_ref[...], w_ref[0], preferred_element_type=jnp.float32)
    y = y + b_ref[0]
    wm = wmat_ref[...]
    col = jax.lax.broadcasted_iota(jnp.int32, wm.shape, 1)
    wcol = jnp.sum(jnp.where(col == e, wm, 0.0), axis=1, keepdims=True)
    acc_ref[...] += wcol * y

    @pl.when(e == ne - 1)
    def _():
        out_ref[...] = acc_ref[...]


def _dense_experts(x2, expert_W, expert_b, wmat, bm, bn):
    T, d = x2.shape
    E = expert_W.shape[0]
    grid = (T // bm, d // bn, E)
    return pl.pallas_call(
        _dense_body,
        grid=grid,
        in_specs=[
            pl.BlockSpec((bm, d), lambda m, n, e: (m, 0)),
            pl.BlockSpec((1, d, bn), lambda m, n, e: (e, 0, n)),
            pl.BlockSpec((1, 1, bn), lambda m, n, e: (e, 0, n)),
            pl.BlockSpec((bm, E), lambda m, n, e: (m, 0)),
        ],
        out_specs=pl.BlockSpec((bm, bn), lambda m, n, e: (m, n)),
        out_shape=jax.ShapeDtypeStruct((T, d), jnp.float32),
        scratch_shapes=[pltpu.VMEM((bm, bn), jnp.float32)],
        compiler_params=pltpu.CompilerParams(
            dimension_semantics=("parallel", "parallel", "arbitrary"),
        ),
    )(x2, expert_W, expert_b.reshape(E, 1, d), wmat)


def kernel(x, gate_W, gate_b, expert_W, expert_b):
    b, s, d = x.shape
    T = b * s
    x2 = x.reshape(T, d)
    bm_r = min(1024, T)
    logits, sel, _w, wmat = _router(x2, gate_W, gate_b, bm_r)
    bm = min(1024, T)
    bn = min(512, d)
    out = _dense_experts(x2, expert_W, expert_b, wmat, bm, bn)
    return out.reshape(b, s, d), logits, sel
